# Initial kernel scaffold; baseline (speedup 1.0000x reference)
import jax
import jax.numpy as jnp
from jax.experimental import pallas as pl


def kernel(x, edge_index, batch, W1, b1, W2, b2, Wm, bm, Wt, bt):
    raise NotImplementedError("write your pallas kernel here")


# speedup vs baseline: 5.8814x; 5.8814x over previous
import functools

import jax
import jax.numpy as jnp
from jax import lax
from jax.experimental import pallas as pl
from jax.experimental.pallas import tpu as pltpu
from jax.experimental.pallas import tpu_sc as plsc

N = 10000
NP = 10240
E = 320000
D = 128
G = 64

NC = 2
NS = 16
NW = NC * NS
RO = NP // NW
TRASH = RO
AROWS = RO + 8

CH = 2000
NCH = E // CH
CAP = 12288
B = 64

R = 1024
NB = NP // R
DEGW = 16


def _sc_mesh():
    return plsc.VectorSubcoreMesh(
        core_axis_name="c", subcore_axis_name="s",
        num_cores=NC, num_subcores=NS)


def _wid():
    return lax.axis_index("c") * NS + lax.axis_index("s")


def _compact_body(src_hbm, dst_hbm, deg_hbm, csrc_hbm, cldst_hbm, ccnt_hbm,
                  sbuf, dbuf, csrc_v, cldst_v, accd, cbuf):
    w = _wid()
    lo = w * RO

    def zero_deg(j, _):
        accd[j] = jnp.zeros((DEGW,), jnp.float32)
        return ()
    lax.fori_loop(0, AROWS, zero_deg, ())

    def chunk(ch, cnt):
        pltpu.sync_copy(src_hbm.at[pl.ds(ch * CH, CH)], sbuf)
        pltpu.sync_copy(dst_hbm.at[pl.ds(ch * CH, CH)], dbuf)

        def vreg(k, cnt):
            d = dbuf[pl.ds(k * 16, 16)]
            s = sbuf[pl.ds(k * 16, 16)]
            m = (d >= lo) & (d < lo + RO) & (cnt < CAP - 2 * B)
            ld = d - lo
            psum = plsc.cumsum(m.astype(jnp.int32))
            pos = cnt + psum - 1
            plsc.store_scatter(cldst_v, [pos], ld, mask=m)
            plsc.store_scatter(csrc_v, [pos], s, mask=m)
            return cnt + jnp.max(psum)
        return lax.fori_loop(0, CH // 16, vreg, cnt)
    cnt = lax.fori_loop(0, NCH, chunk, jnp.int32(0))

    for t in range(B // 16):
        cldst_v[pl.ds(cnt + t * 16, 16)] = jnp.full((16,), TRASH, jnp.int32)
        csrc_v[pl.ds(cnt + t * 16, 16)] = jnp.zeros((16,), jnp.int32)

    e0 = jnp.where(lax.iota(jnp.int32, 16) == 0, 1.0, 0.0)

    def degv(q, _):
        ldv = cldst_v[pl.ds(q * 16, 16)]
        for l in range(16):
            plsc.addupdate(accd.at[ldv[l]], e0)
        return ()
    lax.fori_loop(0, (cnt + 15) // 16, degv, ())

    pltpu.sync_copy(accd.at[pl.ds(0, RO)], deg_hbm.at[pl.ds(w * RO, RO)])
    pltpu.sync_copy(csrc_v, csrc_hbm.at[pl.ds(w * CAP, CAP)])
    pltpu.sync_copy(cldst_v, cldst_hbm.at[pl.ds(w * CAP, CAP)])
    cbuf[pl.ds(0, 16)] = jnp.full((16,), cnt, jnp.int32)
    pltpu.sync_copy(cbuf, ccnt_hbm.at[pl.ds(w * 16, 16)])


def _make_compact():
    return functools.partial(
        pl.kernel,
        out_type=(
            jax.ShapeDtypeStruct((NP, DEGW), jnp.float32),
            jax.ShapeDtypeStruct((NW * CAP,), jnp.int32),
            jax.ShapeDtypeStruct((NW * CAP,), jnp.int32),
            jax.ShapeDtypeStruct((NW * 16,), jnp.int32),
        ),
        mesh=_sc_mesh(),
        scratch_types=[
            pltpu.VMEM((CH,), jnp.int32),
            pltpu.VMEM((CH,), jnp.int32),
            pltpu.VMEM((CAP,), jnp.int32),
            pltpu.VMEM((CAP,), jnp.int32),
            pltpu.VMEM((AROWS, DEGW), jnp.float32),
            pltpu.VMEM((16,), jnp.int32),
        ],
        compiler_params=pltpu.CompilerParams(needs_layout_passes=False),
    )(_compact_body)


_compact_sc = _make_compact()


def _agg_body(u_hbm, csrc_hbm, cldst_hbm, ccnt_hbm, out_hbm,
              csrc_v, cldst_v, rows_v, acc, cbuf, sem):
    w = _wid()

    pltpu.sync_copy(ccnt_hbm.at[pl.ds(w * 16, 16)], cbuf)
    cnt = jnp.max(cbuf[...])
    pltpu.sync_copy(csrc_hbm.at[pl.ds(w * CAP, CAP)], csrc_v)
    pltpu.sync_copy(cldst_hbm.at[pl.ds(w * CAP, CAP)], cldst_v)

    def zero_acc(j, _):
        for cc in range(D // 16):
            acc[j, pl.ds(cc * 16, 16)] = jnp.zeros((16,), jnp.float32)
        return ()
    lax.fori_loop(0, AROWS, zero_acc, ())

    def batch(ib, _):
        pltpu.async_copy(u_hbm.at[csrc_v.at[pl.ds(ib * B, B)]],
                         rows_v, sem).wait()

        def sub(q, _):
            ldv = cldst_v[pl.ds(ib * B + q * 16, 16)]
            for l in range(16):
                r = ldv[l]
                for cc in range(D // 16):
                    plsc.addupdate(
                        acc.at[r, pl.ds(cc * 16, 16)],
                        rows_v[q * 16 + l, pl.ds(cc * 16, 16)])
            return ()
        lax.fori_loop(0, B // 16, sub, ())
        return ()
    lax.fori_loop(0, (cnt + B - 1) // B, batch, ())

    pltpu.sync_copy(acc.at[pl.ds(0, RO)], out_hbm.at[pl.ds(w * RO, RO)])


def _make_agg():
    return functools.partial(
        pl.kernel,
        out_type=jax.ShapeDtypeStruct((NP, D), jnp.float32),
        mesh=_sc_mesh(),
        scratch_types=[
            pltpu.VMEM((CAP,), jnp.int32),
            pltpu.VMEM((CAP,), jnp.int32),
            pltpu.VMEM((B, D), jnp.float32),
            pltpu.VMEM((AROWS, D), jnp.float32),
            pltpu.VMEM((16,), jnp.int32),
            pltpu.SemaphoreType.DMA,
        ],
        compiler_params=pltpu.CompilerParams(needs_layout_passes=False),
    )(_agg_body)


_agg_sc = _make_agg()


def _k2(deg, x, W1):
    def body(d0, x_ref, w_ref, u_ref, dinv_ref):
        dinv = lax.rsqrt(d0[:, 0:1] + 1.0)
        h = jnp.dot(x_ref[...], w_ref[...], preferred_element_type=jnp.float32)
        u_ref[...] = h * dinv
        dinv_ref[...] = dinv
    return pl.pallas_call(
        body,
        grid=(NB,),
        in_specs=[
            pl.BlockSpec((R, DEGW), lambda i: (i, 0)),
            pl.BlockSpec((R, D), lambda i: (i, 0)),
            pl.BlockSpec((D, D), lambda i: (0, 0)),
        ],
        out_specs=[
            pl.BlockSpec((R, D), lambda i: (i, 0)),
            pl.BlockSpec((R, 1), lambda i: (i, 0)),
        ],
        out_shape=[
            jax.ShapeDtypeStruct((NP, D), jnp.float32),
            jax.ShapeDtypeStruct((NP, 1), jnp.float32),
        ],
    )(deg, x, W1)


def _k4(agg, u, dinv, b, W):
    def body(a_ref, u_ref, dinv_ref, b_ref, w_ref, o_ref):
        dinv_b = dinv_ref[...]
        xin = jnp.maximum(
            dinv_b * (a_ref[...] + u_ref[...]) + b_ref[...], 0.0)
        h = jnp.dot(xin, w_ref[...], preferred_element_type=jnp.float32)
        o_ref[...] = h * dinv_b
    return pl.pallas_call(
        body,
        grid=(NB,),
        in_specs=[
            pl.BlockSpec((R, D), lambda i: (i, 0)),
            pl.BlockSpec((R, D), lambda i: (i, 0)),
            pl.BlockSpec((R, 1), lambda i: (i, 0)),
            pl.BlockSpec((1, D), lambda i: (0, 0)),
            pl.BlockSpec((D, D), lambda i: (0, 0)),
        ],
        out_specs=pl.BlockSpec((R, D), lambda i: (i, 0)),
        out_shape=jax.ShapeDtypeStruct((NP, D), jnp.float32),
    )(agg, u, dinv, b, W)


def _k6(agg, u, dinv, b, batch, Wm, bm, Wt, bt):
    def body(a_ref, u_ref, dinv_ref, b_ref, batch_ref,
             wm_ref, bm_ref, wt_ref, bt_ref, mem_ref, time_ref,
             sums, cnts):
        i = pl.program_id(0)

        @pl.when(i == 0)
        def _():
            sums[...] = jnp.zeros_like(sums)
            cnts[...] = jnp.zeros_like(cnts)

        dinv_b = dinv_ref[...]
        h = jnp.maximum(
            dinv_b * (a_ref[...] + u_ref[...]) + b_ref[...], 0.0)
        oh = (batch_ref[...] ==
              lax.broadcasted_iota(jnp.int32, (R, G), 1)).astype(jnp.float32)
        sums[...] += lax.dot_general(
            oh, h, (((0,), (0,)), ((), ())),
            preferred_element_type=jnp.float32)
        cnts[...] += lax.dot_general(
            oh, jnp.ones((R, D), jnp.float32), (((0,), (0,)), ((), ())),
            preferred_element_type=jnp.float32)

        @pl.when(i == NB - 1)
        def _():
            mean = sums[...] / jnp.maximum(cnts[...], 1.0)
            mem_ref[...] = jnp.dot(
                mean, wm_ref[...],
                preferred_element_type=jnp.float32) + bm_ref[...]
            time_ref[...] = jnp.dot(
                mean, wt_ref[...],
                preferred_element_type=jnp.float32) + bt_ref[...]

    return pl.pallas_call(
        body,
        grid=(NB,),
        in_specs=[
            pl.BlockSpec((R, D), lambda i: (i, 0)),
            pl.BlockSpec((R, D), lambda i: (i, 0)),
            pl.BlockSpec((R, 1), lambda i: (i, 0)),
            pl.BlockSpec((1, D), lambda i: (0, 0)),
            pl.BlockSpec((R, 1), lambda i: (i, 0)),
            pl.BlockSpec((D, 1), lambda i: (0, 0)),
            pl.BlockSpec((1, 1), lambda i: (0, 0)),
            pl.BlockSpec((D, 1), lambda i: (0, 0)),
            pl.BlockSpec((1, 1), lambda i: (0, 0)),
        ],
        out_specs=[
            pl.BlockSpec((G, 1), lambda i: (0, 0)),
            pl.BlockSpec((G, 1), lambda i: (0, 0)),
        ],
        out_shape=[
            jax.ShapeDtypeStruct((G, 1), jnp.float32),
            jax.ShapeDtypeStruct((G, 1), jnp.float32),
        ],
        scratch_shapes=[
            pltpu.VMEM((G, D), jnp.float32),
            pltpu.VMEM((G, D), jnp.float32),
        ],
    )(agg, u, dinv, b, batch, Wm, bm, Wt, bt)


def kernel(x, edge_index, batch, W1, b1, W2, b2, Wm, bm, Wt, bt):
    src = edge_index[0]
    dst = edge_index[1]
    x_pad = jnp.pad(x, ((0, NP - N), (0, 0)))
    batch_pad = jnp.pad(batch.reshape(N, 1), ((0, NP - N), (0, 0)),
                        constant_values=G)

    deg, csrc, cldst, ccnt = _compact_sc(src, dst)
    u1, dinv = _k2(deg, x_pad, W1)
    a1 = _agg_sc(u1, csrc, cldst, ccnt)
    u2 = _k4(a1, u1, dinv, b1.reshape(1, D), W2)
    a2 = _agg_sc(u2, csrc, cldst, ccnt)
    mem, tim = _k6(a2, u2, dinv, b2.reshape(1, D), batch_pad,
                   Wm, bm.reshape(1, 1), Wt, bt.reshape(1, 1))
    return mem.reshape(G), tim.reshape(G)


# speedup vs baseline: 6.8152x; 1.1588x over previous
import functools

import jax
import jax.numpy as jnp
from jax import lax
from jax.experimental import pallas as pl
from jax.experimental.pallas import tpu as pltpu
from jax.experimental.pallas import tpu_sc as plsc

N = 10000
NP = 10240
E = 320000
D = 128
G = 64

NC = 2
NS = 16
NW = NC * NS
RO = NP // NW
TRASH = RO
AROWS = RO + 8

CH = 2000
NCH = E // CH
CAP = 12288
B = 128

R = 1024
NB = NP // R
DEGW = 16


def _sc_mesh():
    return plsc.VectorSubcoreMesh(
        core_axis_name="c", subcore_axis_name="s",
        num_cores=NC, num_subcores=NS)


def _wid():
    return lax.axis_index("c") * NS + lax.axis_index("s")


def _compact_body(src_hbm, dst_hbm, deg_hbm, csrc_hbm, cldst_hbm, ccnt_hbm,
                  sbuf, dbuf, csrc_v, cldst_v, accd, cbuf):
    w = _wid()
    lo = w * RO

    def zero_deg(j, _):
        accd[j] = jnp.zeros((DEGW,), jnp.float32)
        return ()
    lax.fori_loop(0, AROWS, zero_deg, ())

    def chunk(ch, cnt):
        pltpu.sync_copy(src_hbm.at[pl.ds(ch * CH, CH)], sbuf)
        pltpu.sync_copy(dst_hbm.at[pl.ds(ch * CH, CH)], dbuf)

        def vreg(k, cnt):
            d = dbuf[pl.ds(k * 16, 16)]
            s = sbuf[pl.ds(k * 16, 16)]
            m = (d >= lo) & (d < lo + RO) & (cnt < CAP - 2 * B)
            ld = d - lo
            psum = plsc.cumsum(m.astype(jnp.int32))
            pos = cnt + psum - 1
            plsc.store_scatter(cldst_v, [pos], ld, mask=m)
            plsc.store_scatter(csrc_v, [pos], s, mask=m)
            return cnt + jnp.max(psum)
        return lax.fori_loop(0, CH // 16, vreg, cnt)
    cnt = lax.fori_loop(0, NCH, chunk, jnp.int32(0))

    for t in range(B // 16):
        cldst_v[pl.ds(cnt + t * 16, 16)] = jnp.full((16,), TRASH, jnp.int32)
        csrc_v[pl.ds(cnt + t * 16, 16)] = jnp.zeros((16,), jnp.int32)

    e0 = jnp.where(lax.iota(jnp.int32, 16) == 0, 1.0, 0.0)

    def degv(q, _):
        ldv = cldst_v[pl.ds(q * 16, 16)]
        for l in range(16):
            plsc.addupdate(accd.at[ldv[l]], e0)
        return ()
    lax.fori_loop(0, (cnt + 15) // 16, degv, ())

    pltpu.sync_copy(accd.at[pl.ds(0, RO)], deg_hbm.at[pl.ds(w * RO, RO)])
    pltpu.sync_copy(csrc_v, csrc_hbm.at[pl.ds(w * CAP, CAP)])
    pltpu.sync_copy(cldst_v, cldst_hbm.at[pl.ds(w * CAP, CAP)])
    cbuf[pl.ds(0, 16)] = jnp.full((16,), cnt, jnp.int32)
    pltpu.sync_copy(cbuf, ccnt_hbm.at[pl.ds(w * 16, 16)])


def _make_compact():
    return functools.partial(
        pl.kernel,
        out_type=(
            jax.ShapeDtypeStruct((NP, DEGW), jnp.float32),
            jax.ShapeDtypeStruct((NW * CAP,), jnp.int32),
            jax.ShapeDtypeStruct((NW * CAP,), jnp.int32),
            jax.ShapeDtypeStruct((NW * 16,), jnp.int32),
        ),
        mesh=_sc_mesh(),
        scratch_types=[
            pltpu.VMEM((CH,), jnp.int32),
            pltpu.VMEM((CH,), jnp.int32),
            pltpu.VMEM((CAP,), jnp.int32),
            pltpu.VMEM((CAP,), jnp.int32),
            pltpu.VMEM((AROWS, DEGW), jnp.float32),
            pltpu.VMEM((16,), jnp.int32),
        ],
        compiler_params=pltpu.CompilerParams(needs_layout_passes=False),
    )(_compact_body)


_compact_sc = _make_compact()


def _agg_body(u_hbm, csrc_hbm, cldst_hbm, ccnt_hbm, out_hbm,
              csrc_v, cldst_v, rows0, rows1, acc, cbuf, sem0, sem1):
    w = _wid()

    pltpu.sync_copy(ccnt_hbm.at[pl.ds(w * 16, 16)], cbuf)
    cnt = jnp.max(cbuf[...])
    pltpu.sync_copy(csrc_hbm.at[pl.ds(w * CAP, CAP)], csrc_v)
    pltpu.sync_copy(cldst_hbm.at[pl.ds(w * CAP, CAP)], cldst_v)
    nb = (cnt + B - 1) // B

    def zero_acc(j, _):
        for cc in range(D // 16):
            acc[j, pl.ds(cc * 16, 16)] = jnp.zeros((16,), jnp.float32)
        return ()
    lax.fori_loop(0, AROWS, zero_acc, ())

    @pl.when(0 < nb)
    def _():
        pltpu.async_copy(u_hbm.at[csrc_v.at[pl.ds(0, B)]], rows0, sem0)

    @pl.when(1 < nb)
    def _():
        pltpu.async_copy(u_hbm.at[csrc_v.at[pl.ds(B, B)]], rows1, sem1)

    def pair(g, _):
        for b in range(2):
            rv = (rows0, rows1)[b]
            sm = (sem0, sem1)[b]
            ib = g * 2 + b

            @pl.when(ib < nb)
            def _(ib=ib, rv=rv, sm=sm):
                pltpu.make_async_copy(
                    u_hbm.at[csrc_v.at[pl.ds(ib * B, B)]], rv, sm).wait()

                def sub(q, _):
                    ldv = cldst_v[pl.ds(ib * B + q * 16, 16)]
                    for l in range(16):
                        r = ldv[l]
                        for cc in range(D // 16):
                            plsc.addupdate(
                                acc.at[r, pl.ds(cc * 16, 16)],
                                rv[q * 16 + l, pl.ds(cc * 16, 16)])
                    return ()
                lax.fori_loop(0, B // 16, sub, ())

                @pl.when(ib + 2 < nb)
                def _():
                    pltpu.async_copy(
                        u_hbm.at[csrc_v.at[pl.ds((ib + 2) * B, B)]], rv, sm)
        return ()
    lax.fori_loop(0, (nb + 1) // 2, pair, ())

    pltpu.sync_copy(acc.at[pl.ds(0, RO)], out_hbm.at[pl.ds(w * RO, RO)])


def _make_agg():
    return functools.partial(
        pl.kernel,
        out_type=jax.ShapeDtypeStruct((NP, D), jnp.float32),
        mesh=_sc_mesh(),
        scratch_types=[
            pltpu.VMEM((CAP,), jnp.int32),
            pltpu.VMEM((CAP,), jnp.int32),
            pltpu.VMEM((B, D), jnp.float32),
            pltpu.VMEM((B, D), jnp.float32),
            pltpu.VMEM((AROWS, D), jnp.float32),
            pltpu.VMEM((16,), jnp.int32),
            pltpu.SemaphoreType.DMA,
            pltpu.SemaphoreType.DMA,
        ],
        compiler_params=pltpu.CompilerParams(needs_layout_passes=False),
    )(_agg_body)


_agg_sc = _make_agg()


def _k2(deg, x, W1):
    def body(d0, x_ref, w_ref, u_ref, dinv_ref):
        dinv = lax.rsqrt(d0[:, 0:1] + 1.0)
        h = jnp.dot(x_ref[...], w_ref[...], preferred_element_type=jnp.float32)
        u_ref[...] = h * dinv
        dinv_ref[...] = dinv
    return pl.pallas_call(
        body,
        grid=(NB,),
        in_specs=[
            pl.BlockSpec((R, DEGW), lambda i: (i, 0)),
            pl.BlockSpec((R, D), lambda i: (i, 0)),
            pl.BlockSpec((D, D), lambda i: (0, 0)),
        ],
        out_specs=[
            pl.BlockSpec((R, D), lambda i: (i, 0)),
            pl.BlockSpec((R, 1), lambda i: (i, 0)),
        ],
        out_shape=[
            jax.ShapeDtypeStruct((NP, D), jnp.float32),
            jax.ShapeDtypeStruct((NP, 1), jnp.float32),
        ],
    )(deg, x, W1)


def _k4(agg, u, dinv, b, W):
    def body(a_ref, u_ref, dinv_ref, b_ref, w_ref, o_ref):
        dinv_b = dinv_ref[...]
        xin = jnp.maximum(
            dinv_b * (a_ref[...] + u_ref[...]) + b_ref[...], 0.0)
        h = jnp.dot(xin, w_ref[...], preferred_element_type=jnp.float32)
        o_ref[...] = h * dinv_b
    return pl.pallas_call(
        body,
        grid=(NB,),
        in_specs=[
            pl.BlockSpec((R, D), lambda i: (i, 0)),
            pl.BlockSpec((R, D), lambda i: (i, 0)),
            pl.BlockSpec((R, 1), lambda i: (i, 0)),
            pl.BlockSpec((1, D), lambda i: (0, 0)),
            pl.BlockSpec((D, D), lambda i: (0, 0)),
        ],
        out_specs=pl.BlockSpec((R, D), lambda i: (i, 0)),
        out_shape=jax.ShapeDtypeStruct((NP, D), jnp.float32),
    )(agg, u, dinv, b, W)


def _k6(agg, u, dinv, b, batch, Wm, bm, Wt, bt):
    def body(a_ref, u_ref, dinv_ref, b_ref, batch_ref,
             wm_ref, bm_ref, wt_ref, bt_ref, mem_ref, time_ref,
             sums, cnts):
        i = pl.program_id(0)

        @pl.when(i == 0)
        def _():
            sums[...] = jnp.zeros_like(sums)
            cnts[...] = jnp.zeros_like(cnts)

        dinv_b = dinv_ref[...]
        h = jnp.maximum(
            dinv_b * (a_ref[...] + u_ref[...]) + b_ref[...], 0.0)
        oh = (batch_ref[...] ==
              lax.broadcasted_iota(jnp.int32, (R, G), 1)).astype(jnp.float32)
        sums[...] += lax.dot_general(
            oh, h, (((0,), (0,)), ((), ())),
            preferred_element_type=jnp.float32)
        cnts[...] += lax.dot_general(
            oh, jnp.ones((R, D), jnp.float32), (((0,), (0,)), ((), ())),
            preferred_element_type=jnp.float32)

        @pl.when(i == NB - 1)
        def _():
            mean = sums[...] / jnp.maximum(cnts[...], 1.0)
            mem_ref[...] = jnp.dot(
                mean, wm_ref[...],
                preferred_element_type=jnp.float32) + bm_ref[...]
            time_ref[...] = jnp.dot(
                mean, wt_ref[...],
                preferred_element_type=jnp.float32) + bt_ref[...]

    return pl.pallas_call(
        body,
        grid=(NB,),
        in_specs=[
            pl.BlockSpec((R, D), lambda i: (i, 0)),
            pl.BlockSpec((R, D), lambda i: (i, 0)),
            pl.BlockSpec((R, 1), lambda i: (i, 0)),
            pl.BlockSpec((1, D), lambda i: (0, 0)),
            pl.BlockSpec((R, 1), lambda i: (i, 0)),
            pl.BlockSpec((D, 1), lambda i: (0, 0)),
            pl.BlockSpec((1, 1), lambda i: (0, 0)),
            pl.BlockSpec((D, 1), lambda i: (0, 0)),
            pl.BlockSpec((1, 1), lambda i: (0, 0)),
        ],
        out_specs=[
            pl.BlockSpec((G, 1), lambda i: (0, 0)),
            pl.BlockSpec((G, 1), lambda i: (0, 0)),
        ],
        out_shape=[
            jax.ShapeDtypeStruct((G, 1), jnp.float32),
            jax.ShapeDtypeStruct((G, 1), jnp.float32),
        ],
        scratch_shapes=[
            pltpu.VMEM((G, D), jnp.float32),
            pltpu.VMEM((G, D), jnp.float32),
        ],
    )(agg, u, dinv, b, batch, Wm, bm, Wt, bt)


def kernel(x, edge_index, batch, W1, b1, W2, b2, Wm, bm, Wt, bt):
    src = edge_index[0]
    dst = edge_index[1]
    x_pad = jnp.pad(x, ((0, NP - N), (0, 0)))
    batch_pad = jnp.pad(batch.reshape(N, 1), ((0, NP - N), (0, 0)),
                        constant_values=G)

    deg, csrc, cldst, ccnt = _compact_sc(src, dst)
    u1, dinv = _k2(deg, x_pad, W1)
    a1 = _agg_sc(u1, csrc, cldst, ccnt)
    u2 = _k4(a1, u1, dinv, b1.reshape(1, D), W2)
    a2 = _agg_sc(u2, csrc, cldst, ccnt)
    mem, tim = _k6(a2, u2, dinv, b2.reshape(1, D), batch_pad,
                   Wm, bm.reshape(1, 1), Wt, bt.reshape(1, 1))
    return mem.reshape(G), tim.reshape(G)


# speedup vs baseline: 10.6840x; 1.5677x over previous
import functools

import jax
import jax.numpy as jnp
from jax import lax
from jax.experimental import pallas as pl
from jax.experimental.pallas import tpu as pltpu
from jax.experimental.pallas import tpu_sc as plsc

N = 10000
NP = 10240
E = 320000
D = 128
G = 64

NC = 2
NS = 16
NW = NC * NS
RO = NP // NW
TRASH = RO
AROWS = RO + 8

CH = 3200
NCH = E // CH
CAP = 12288
B = 128

R = 1024
NB = NP // R
DEGW = 16


def _sc_mesh():
    return plsc.VectorSubcoreMesh(
        core_axis_name="c", subcore_axis_name="s",
        num_cores=NC, num_subcores=NS)


def _wid():
    return lax.axis_index("c") * NS + lax.axis_index("s")


def _compact_body(ei_hbm, deg_hbm, csrc_hbm, cldst_hbm, ccnt_hbm,
                  ebuf0, ebuf1, csrc_v, cldst_v, accd, cbuf, sem0, sem1):
    w = _wid()
    lo = w * RO

    def zero_deg(j, _):
        accd[j] = jnp.zeros((DEGW,), jnp.float32)
        return ()
    lax.fori_loop(0, AROWS, zero_deg, ())

    pltpu.async_copy(ei_hbm.at[:, pl.ds(0, CH)], ebuf0, sem0)
    pltpu.async_copy(ei_hbm.at[:, pl.ds(CH, CH)], ebuf1, sem1)

    def scan_chunk(ebuf, cnt):
        def vreg4(k4, cnt):
            ok = cnt < CAP - 2 * B - 64
            ms, lds, ss, pcs, pss = [], [], [], [], []
            for t in range(4):
                off = k4 * 64 + t * 16
                d = ebuf[1, pl.ds(off, 16)]
                s = ebuf[0, pl.ds(off, 16)]
                m = (d >= lo) & (d < lo + RO) & ok
                pcs.append(plsc.all_reduce_population_count(m))
                pss.append(plsc.cumsum(m.astype(jnp.int32)))
                ms.append(m)
                lds.append(d - lo)
                ss.append(s)
            base = cnt
            for t in range(4):
                pos = base + pss[t] - 1
                plsc.store_scatter(cldst_v, [pos], lds[t], mask=ms[t])
                plsc.store_scatter(csrc_v, [pos], ss[t], mask=ms[t])
                base = base + pcs[t][0]
            return base
        return lax.fori_loop(0, CH // 64, vreg4, cnt)

    def chunk_pair(g, cnt):
        for b in range(2):
            ch = g * 2 + b
            ebuf = (ebuf0, ebuf1)[b]
            sm = (sem0, sem1)[b]
            pltpu.make_async_copy(
                ei_hbm.at[:, pl.ds(ch * CH, CH)], ebuf, sm).wait()
            cnt = scan_chunk(ebuf, cnt)

            @pl.when(ch + 2 < NCH)
            def _(ebuf=ebuf, sm=sm, ch=ch):
                pltpu.async_copy(
                    ei_hbm.at[:, pl.ds((ch + 2) * CH, CH)], ebuf, sm)
        return cnt
    cnt = lax.fori_loop(0, NCH // 2, chunk_pair, jnp.int32(0))

    for t in range(B // 16):
        cldst_v[pl.ds(cnt + t * 16, 16)] = jnp.full((16,), TRASH, jnp.int32)
        csrc_v[pl.ds(cnt + t * 16, 16)] = jnp.zeros((16,), jnp.int32)

    e0 = jnp.where(lax.iota(jnp.int32, 16) == 0, 1.0, 0.0)

    def degv(q, _):
        ldv = cldst_v[pl.ds(q * 16, 16)]
        for l in range(16):
            plsc.addupdate(accd.at[ldv[l]], e0)
        return ()
    lax.fori_loop(0, (cnt + 15) // 16, degv, ())

    pltpu.sync_copy(accd.at[pl.ds(0, RO)], deg_hbm.at[pl.ds(w * RO, RO)])
    pltpu.sync_copy(csrc_v, csrc_hbm.at[pl.ds(w * CAP, CAP)])
    pltpu.sync_copy(cldst_v, cldst_hbm.at[pl.ds(w * CAP, CAP)])
    cbuf[pl.ds(0, 16)] = jnp.full((16,), cnt, jnp.int32)
    pltpu.sync_copy(cbuf, ccnt_hbm.at[pl.ds(w * 16, 16)])


def _make_compact():
    return functools.partial(
        pl.kernel,
        out_type=(
            jax.ShapeDtypeStruct((NP, DEGW), jnp.float32),
            jax.ShapeDtypeStruct((NW * CAP,), jnp.int32),
            jax.ShapeDtypeStruct((NW * CAP,), jnp.int32),
            jax.ShapeDtypeStruct((NW * 16,), jnp.int32),
        ),
        mesh=_sc_mesh(),
        scratch_types=[
            pltpu.VMEM((2, CH), jnp.int32),
            pltpu.VMEM((2, CH), jnp.int32),
            pltpu.VMEM((CAP,), jnp.int32),
            pltpu.VMEM((CAP,), jnp.int32),
            pltpu.VMEM((AROWS, DEGW), jnp.float32),
            pltpu.VMEM((16,), jnp.int32),
            pltpu.SemaphoreType.DMA,
            pltpu.SemaphoreType.DMA,
        ],
        compiler_params=pltpu.CompilerParams(needs_layout_passes=False),
    )(_compact_body)


_compact_sc = _make_compact()


def _agg_body(u_hbm, csrc_hbm, cldst_hbm, ccnt_hbm, out_hbm,
              csrc_v, cldst_v, rows0, rows1, acc, cbuf, sem0, sem1):
    w = _wid()

    pltpu.sync_copy(ccnt_hbm.at[pl.ds(w * 16, 16)], cbuf)
    cnt = jnp.max(cbuf[...])
    pltpu.sync_copy(csrc_hbm.at[pl.ds(w * CAP, CAP)], csrc_v)
    pltpu.sync_copy(cldst_hbm.at[pl.ds(w * CAP, CAP)], cldst_v)
    nb = (cnt + B - 1) // B

    def zero_acc(j, _):
        for cc in range(D // 16):
            acc[j, pl.ds(cc * 16, 16)] = jnp.zeros((16,), jnp.float32)
        return ()
    lax.fori_loop(0, AROWS, zero_acc, ())

    @pl.when(0 < nb)
    def _():
        pltpu.async_copy(u_hbm.at[csrc_v.at[pl.ds(0, B)]], rows0, sem0)

    @pl.when(1 < nb)
    def _():
        pltpu.async_copy(u_hbm.at[csrc_v.at[pl.ds(B, B)]], rows1, sem1)

    def pair(g, _):
        for b in range(2):
            rv = (rows0, rows1)[b]
            sm = (sem0, sem1)[b]
            ib = g * 2 + b

            @pl.when(ib < nb)
            def _(ib=ib, rv=rv, sm=sm):
                pltpu.make_async_copy(
                    u_hbm.at[csrc_v.at[pl.ds(ib * B, B)]], rv, sm).wait()

                def sub(q, _):
                    ldv = cldst_v[pl.ds(ib * B + q * 16, 16)]
                    for l in range(16):
                        r = ldv[l]
                        for cc in range(D // 16):
                            plsc.addupdate(
                                acc.at[r, pl.ds(cc * 16, 16)],
                                rv[q * 16 + l, pl.ds(cc * 16, 16)])
                    return ()
                lax.fori_loop(0, B // 16, sub, ())

                @pl.when(ib + 2 < nb)
                def _():
                    pltpu.async_copy(
                        u_hbm.at[csrc_v.at[pl.ds((ib + 2) * B, B)]], rv, sm)
        return ()
    lax.fori_loop(0, (nb + 1) // 2, pair, ())

    pltpu.sync_copy(acc.at[pl.ds(0, RO)], out_hbm.at[pl.ds(w * RO, RO)])


def _make_agg():
    return functools.partial(
        pl.kernel,
        out_type=jax.ShapeDtypeStruct((NP, D), jnp.float32),
        mesh=_sc_mesh(),
        scratch_types=[
            pltpu.VMEM((CAP,), jnp.int32),
            pltpu.VMEM((CAP,), jnp.int32),
            pltpu.VMEM((B, D), jnp.float32),
            pltpu.VMEM((B, D), jnp.float32),
            pltpu.VMEM((AROWS, D), jnp.float32),
            pltpu.VMEM((16,), jnp.int32),
            pltpu.SemaphoreType.DMA,
            pltpu.SemaphoreType.DMA,
        ],
        compiler_params=pltpu.CompilerParams(needs_layout_passes=False),
    )(_agg_body)


_agg_sc = _make_agg()


def _k2(deg, x, W1):
    def body(d0, x_ref, w_ref, u_ref, dinv_ref):
        dinv = lax.rsqrt(d0[:, 0:1] + 1.0)
        h = jnp.dot(x_ref[...], w_ref[...], preferred_element_type=jnp.float32)
        u_ref[...] = h * dinv
        dinv_ref[...] = dinv
    return pl.pallas_call(
        body,
        grid=(NB,),
        in_specs=[
            pl.BlockSpec((R, DEGW), lambda i: (i, 0)),
            pl.BlockSpec((R, D), lambda i: (i, 0)),
            pl.BlockSpec((D, D), lambda i: (0, 0)),
        ],
        out_specs=[
            pl.BlockSpec((R, D), lambda i: (i, 0)),
            pl.BlockSpec((R, 1), lambda i: (i, 0)),
        ],
        out_shape=[
            jax.ShapeDtypeStruct((NP, D), jnp.float32),
            jax.ShapeDtypeStruct((NP, 1), jnp.float32),
        ],
    )(deg, x, W1)


def _k4(agg, u, dinv, b, W):
    def body(a_ref, u_ref, dinv_ref, b_ref, w_ref, o_ref):
        dinv_b = dinv_ref[...]
        xin = jnp.maximum(
            dinv_b * (a_ref[...] + u_ref[...]) + b_ref[...], 0.0)
        h = jnp.dot(xin, w_ref[...], preferred_element_type=jnp.float32)
        o_ref[...] = h * dinv_b
    return pl.pallas_call(
        body,
        grid=(NB,),
        in_specs=[
            pl.BlockSpec((R, D), lambda i: (i, 0)),
            pl.BlockSpec((R, D), lambda i: (i, 0)),
            pl.BlockSpec((R, 1), lambda i: (i, 0)),
            pl.BlockSpec((1, D), lambda i: (0, 0)),
            pl.BlockSpec((D, D), lambda i: (0, 0)),
        ],
        out_specs=pl.BlockSpec((R, D), lambda i: (i, 0)),
        out_shape=jax.ShapeDtypeStruct((NP, D), jnp.float32),
    )(agg, u, dinv, b, W)


def _k6(agg, u, dinv, b, batch, Wm, bm, Wt, bt):
    def body(a_ref, u_ref, dinv_ref, b_ref, batch_ref,
             wm_ref, bm_ref, wt_ref, bt_ref, mem_ref, time_ref,
             sums, cnts):
        i = pl.program_id(0)

        @pl.when(i == 0)
        def _():
            sums[...] = jnp.zeros_like(sums)
            cnts[...] = jnp.zeros_like(cnts)

        dinv_b = dinv_ref[...]
        h = jnp.maximum(
            dinv_b * (a_ref[...] + u_ref[...]) + b_ref[...], 0.0)
        oh = (batch_ref[...] ==
              lax.broadcasted_iota(jnp.int32, (R, G), 1)).astype(jnp.float32)
        sums[...] += lax.dot_general(
            oh, h, (((0,), (0,)), ((), ())),
            preferred_element_type=jnp.float32)
        cnts[...] += lax.dot_general(
            oh, jnp.ones((R, D), jnp.float32), (((0,), (0,)), ((), ())),
            preferred_element_type=jnp.float32)

        @pl.when(i == NB - 1)
        def _():
            mean = sums[...] / jnp.maximum(cnts[...], 1.0)
            mem_ref[...] = jnp.dot(
                mean, wm_ref[...],
                preferred_element_type=jnp.float32) + bm_ref[...]
            time_ref[...] = jnp.dot(
                mean, wt_ref[...],
                preferred_element_type=jnp.float32) + bt_ref[...]

    return pl.pallas_call(
        body,
        grid=(NB,),
        in_specs=[
            pl.BlockSpec((R, D), lambda i: (i, 0)),
            pl.BlockSpec((R, D), lambda i: (i, 0)),
            pl.BlockSpec((R, 1), lambda i: (i, 0)),
            pl.BlockSpec((1, D), lambda i: (0, 0)),
            pl.BlockSpec((R, 1), lambda i: (i, 0)),
            pl.BlockSpec((D, 1), lambda i: (0, 0)),
            pl.BlockSpec((1, 1), lambda i: (0, 0)),
            pl.BlockSpec((D, 1), lambda i: (0, 0)),
            pl.BlockSpec((1, 1), lambda i: (0, 0)),
        ],
        out_specs=[
            pl.BlockSpec((G, 1), lambda i: (0, 0)),
            pl.BlockSpec((G, 1), lambda i: (0, 0)),
        ],
        out_shape=[
            jax.ShapeDtypeStruct((G, 1), jnp.float32),
            jax.ShapeDtypeStruct((G, 1), jnp.float32),
        ],
        scratch_shapes=[
            pltpu.VMEM((G, D), jnp.float32),
            pltpu.VMEM((G, D), jnp.float32),
        ],
    )(agg, u, dinv, b, batch, Wm, bm, Wt, bt)


def kernel(x, edge_index, batch, W1, b1, W2, b2, Wm, bm, Wt, bt):
    x_pad = jnp.pad(x, ((0, NP - N), (0, 0)))
    batch_pad = jnp.pad(batch.reshape(N, 1), ((0, NP - N), (0, 0)),
                        constant_values=G)

    deg, csrc, cldst, ccnt = _compact_sc(edge_index)
    u1, dinv = _k2(deg, x_pad, W1)
    a1 = _agg_sc(u1, csrc, cldst, ccnt)
    u2 = _k4(a1, u1, dinv, b1.reshape(1, D), W2)
    a2 = _agg_sc(u2, csrc, cldst, ccnt)
    mem, tim = _k6(a2, u2, dinv, b2.reshape(1, D), batch_pad,
                   Wm, bm.reshape(1, 1), Wt, bt.reshape(1, 1))
    return mem.reshape(G), tim.reshape(G)
